# baseline (speedup 1.0000x reference)
import jax
import jax.numpy as jnp
from jax import lax
from jax.experimental import pallas as pl
from jax.experimental.pallas import tpu as pltpu
from jax.experimental.pallas import tpu_sc as plsc

B = 4096
HIST = 50
D = 64
L = 16
NJ = D // L
NC, NS = 2, 16
NW = NC * NS
BW = B // NW
GRP = 2
ROWS = GRP * HIST
ND = BW // GRP
LG = BW // L
HGE = L // 2
DPH = HGE // GRP
NH = BW // HGE
AVG_RATING = 3.5


def _sc_body(user_h, item_h, simt_h, ubias_h, ibias_h, q_h, x_h,
             lut_h, out_h, ubo_h, ibo_h,
             sim_v, simt_v, uidx_v, iidx_v, urow_v, irow_v, ubr_v, ibr_v,
             ub_v, ib_v, q_v, xl_v, zidx_v, x0r_v, n0f_v, scale_v, out_v,
             rows0, rows1, sem_m, sem0, sem1):
    wid = lax.axis_index("c") * NS + lax.axis_index("s")
    base = wid * BW

    pltpu.sync_copy(simt_h.at[:, pl.ds(base, BW)], simt_v)
    pltpu.sync_copy(user_h.at[pl.ds(base, BW)], uidx_v)
    pltpu.sync_copy(item_h.at[pl.ds(base, BW)], iidx_v)
    pltpu.sync_copy(lut_h, xl_v)
    zidx_v[...] = jnp.zeros((L,), jnp.int32)

    for lg in range(LG):
        sl = pl.ds(lg * L, L)
        urow_v[sl] = lax.shift_right_logical(uidx_v[sl], 4)
        irow_v[sl] = lax.shift_right_logical(iidx_v[sl], 4)

    cp_ub = pltpu.async_copy(ubias_h.at[urow_v], ubr_v, sem_m)
    cp_ib = pltpu.async_copy(ibias_h.at[irow_v], ibr_v, sem_m)
    cp_q = pltpu.async_copy(q_h.at[iidx_v], q_v, sem_m)
    cp_x0 = pltpu.async_copy(x_h.at[zidx_v], x0r_v, sem_m)

    lanes0 = jnp.arange(L, dtype=jnp.int32)

    def build_body(c, _):
        f = c * L + lanes0
        b = f // HIST
        n = f - b * HIST
        v = plsc.load_gather(simt_v, [n, b])
        plsc.store_scatter(sim_v, [f // ROWS, f % ROWS], v)
        return 0

    lax.fori_loop(0, (BW * HIST) // L, build_body, 0)

    def fire(h, rows_v, sem):
        for k in range(DPH):
            pltpu.async_copy(x_h.at[sim_v.at[h * DPH + k]],
                             rows_v.at[pl.ds(k * ROWS, ROWS)], sem)

    fire(0, rows0, sem0)
    fire(1, rows1, sem1)

    one = jnp.ones((L,), jnp.int32)

    def n0_body(lg, _):
        sl = pl.ds(lg * L, L)
        nnz = jnp.zeros((L,), jnp.int32)
        for n in range(HIST):
            nnz = nnz + jnp.minimum(simt_v[n, sl], one)
        n0f_v[sl] = (HIST - nnz).astype(jnp.float32)
        scale_v[sl] = plsc.load_gather(xl_v, [nnz])
        return 0

    lax.fori_loop(0, LG, n0_body, 0)

    cp_ub.wait()
    cp_ib.wait()
    cp_q.wait()
    cp_x0.wait()

    lanes = jnp.arange(L, dtype=jnp.int32)

    def do_half(lg, half, rows_v, sem, n0f16, x0, dotv):
        h = 2 * lg + half
        pltpu.make_async_copy(x_h.at[pl.ds(0, HGE * HIST)], rows_v, sem).wait()

        for p in range(DPH):
            def acc_body(n, a):
                r0 = p * ROWS + 2 * n
                r1 = r0 + HIST
                return tuple(
                    [a[j] + (rows_v[r0, pl.ds(j * L, L)]
                             + rows_v[r0 + 1, pl.ds(j * L, L)])
                     for j in range(NJ)]
                    + [a[NJ + j] + (rows_v[r1, pl.ds(j * L, L)]
                                    + rows_v[r1 + 1, pl.ds(j * L, L)])
                       for j in range(NJ)])

            zero = tuple(jnp.zeros((L,), jnp.float32) for _ in range(2 * NJ))
            a = lax.fori_loop(0, HIST // 2, acc_body, zero)

            for e2 in range(2):
                e = half * HGE + 2 * p + e2
                n0f = n0f16[e]
                s = jnp.zeros((L,), jnp.float32)
                for j in range(NJ):
                    t = a[e2 * NJ + j] - n0f * x0[j]
                    s = s + t * q_v[lg * L + e, pl.ds(j * L, L)]
                oneh = (1 - jnp.minimum(jnp.abs(lanes - e), 1)).astype(
                    jnp.float32)
                dotv = dotv + jnp.sum(s) * oneh

        @pl.when(h + 2 < NH)
        def _():
            fire(h + 2, rows_v, sem)

        return dotv

    def main_body(lg, _):
        sl16 = pl.ds(lg * L, L)
        n0f16 = n0f_v[sl16]
        x0 = [x0r_v[0, pl.ds(j * L, L)] for j in range(NJ)]
        dotv = jnp.zeros((L,), jnp.float32)
        dotv = do_half(lg, 0, rows0, sem0, n0f16, x0, dotv)
        dotv = do_half(lg, 1, rows1, sem1, n0f16, x0, dotv)

        bvec = lg * L + lanes
        ub16 = plsc.load_gather(ubr_v, [bvec, uidx_v[sl16] & 15])
        ib16 = plsc.load_gather(ibr_v, [bvec, iidx_v[sl16] & 15])
        ub_v[sl16] = ub16
        ib_v[sl16] = ib16
        out_v[sl16] = AVG_RATING + ub16 + ib16 + dotv * scale_v[sl16]
        return 0

    lax.fori_loop(0, LG, main_body, 0)

    pltpu.sync_copy(out_v, out_h.at[pl.ds(base, BW)])
    pltpu.sync_copy(ub_v, ubo_h.at[pl.ds(base, BW)])
    pltpu.sync_copy(ib_v, ibo_h.at[pl.ds(base, BW)])


@jax.jit
def kernel(user, item, similar_explicit, user_bias, item_bias, item_q, item_x):
    user = user.astype(jnp.int32)
    item = item.astype(jnp.int32)
    simt = similar_explicit.astype(jnp.int32).T
    lut = 1.0 / (jnp.sqrt(jnp.arange(D, dtype=jnp.float32)) + 1e-13)

    f32, i32 = jnp.float32, jnp.int32
    run = pl.kernel(
        _sc_body,
        out_type=(
            jax.ShapeDtypeStruct((B,), f32),
            jax.ShapeDtypeStruct((B,), f32),
            jax.ShapeDtypeStruct((B,), f32),
        ),
        mesh=plsc.VectorSubcoreMesh(core_axis_name="c", subcore_axis_name="s"),
        compiler_params=pltpu.CompilerParams(use_tc_tiling_on_sc=False,
                                             needs_layout_passes=False),
        scratch_types=(
            pltpu.VMEM((ND, ROWS), i32),
            pltpu.VMEM((HIST, BW), i32),
            pltpu.VMEM((BW,), i32),
            pltpu.VMEM((BW,), i32),
            pltpu.VMEM((BW,), i32),
            pltpu.VMEM((BW,), i32),
            pltpu.VMEM((BW, L), f32),
            pltpu.VMEM((BW, L), f32),
            pltpu.VMEM((BW,), f32),
            pltpu.VMEM((BW,), f32),
            pltpu.VMEM((BW, D), f32),
            pltpu.VMEM((D,), f32),
            pltpu.VMEM((L,), i32),
            pltpu.VMEM((L, D), f32),
            pltpu.VMEM((BW,), f32),
            pltpu.VMEM((BW,), f32),
            pltpu.VMEM((BW,), f32),
            pltpu.VMEM((HGE * HIST, D), f32),
            pltpu.VMEM((HGE * HIST, D), f32),
            pltpu.SemaphoreType.DMA,
            pltpu.SemaphoreType.DMA,
            pltpu.SemaphoreType.DMA,
        ),
    )
    out, ub, ib = run(user, item, simt,
                      user_bias.reshape(-1, L), item_bias.reshape(-1, L),
                      item_q, item_x, lut)
    return (out, ub, ib)


# speedup vs baseline: 1.2045x; 1.2045x over previous
import jax
import jax.numpy as jnp
from jax import lax
from jax.experimental import pallas as pl
from jax.experimental.pallas import tpu as pltpu
from jax.experimental.pallas import tpu_sc as plsc

B = 4096
HIST = 50
D = 64
L = 16
NJ = D // L
NC, NS = 2, 16
NW = NC * NS
BW = B // NW
GRP = 2
ROWS = GRP * HIST
ND = BW // GRP
LG = BW // L
HGE = L // 2
DPH = HGE // GRP
NH = BW // HGE
AVG_RATING = 3.5
XN = 100000
FB = 2048
FG = 25
HALF = FG * FB


def _tc_fold_body(a_ref, b_ref, o_ref):
    o_ref[...] = jnp.concatenate([a_ref[...].T, b_ref[...].T], axis=1)


def _fold_pairs(t):
    tt = t.T
    return pl.pallas_call(
        _tc_fold_body,
        grid=(FG,),
        in_specs=[pl.BlockSpec((D, FB), lambda i: (0, i)),
                  pl.BlockSpec((D, FB),
                               lambda i: (0, jnp.minimum(i + FG, 48)))],
        out_specs=pl.BlockSpec((FB, 2 * D), lambda i: (i, 0)),
        out_shape=jax.ShapeDtypeStruct((HALF, 2 * D), jnp.float32),
    )(tt, tt)


def _sc_body(user_h, item_h, simt_h, ubias_h, ibias_h, q_h, x_h,
             lut_h, out_h, ubo_h, ibo_h,
             sim_v, simt_v, uidx_v, iidx_v, urow_v, irow_v, qrow_v, ubr_v,
             ibr_v,
             ub_v, ib_v, q_v, xl_v, zidx_v, x0r_v, n0f_v, scale_v, out_v,
             rows0, rows1, sem_m, sem0, sem1):
    wid = lax.axis_index("c") * NS + lax.axis_index("s")
    base = wid * BW

    pltpu.sync_copy(simt_h.at[:, pl.ds(base, BW)], simt_v)
    pltpu.sync_copy(user_h.at[pl.ds(base, BW)], uidx_v)
    pltpu.sync_copy(item_h.at[pl.ds(base, BW)], iidx_v)
    pltpu.sync_copy(lut_h, xl_v)
    zidx_v[...] = jnp.zeros((L,), jnp.int32)

    for lg in range(LG):
        sl = pl.ds(lg * L, L)
        urow_v[sl] = lax.shift_right_logical(uidx_v[sl], 4)
        irow_v[sl] = lax.shift_right_logical(iidx_v[sl], 4)
        iv = iidx_v[sl]
        hv = iv // HALF
        qrow_v[sl] = 2 * (iv - HALF * hv) + hv

    cp_ub = pltpu.async_copy(ubias_h.at[urow_v], ubr_v, sem_m)
    cp_ib = pltpu.async_copy(ibias_h.at[irow_v], ibr_v, sem_m)
    cp_q = pltpu.async_copy(q_h.at[qrow_v], q_v, sem_m)
    cp_x0 = pltpu.async_copy(x_h.at[zidx_v], x0r_v, sem_m)

    lanes0 = jnp.arange(L, dtype=jnp.int32)

    def build_body(c, _):
        f = c * L + lanes0
        b = f // HIST
        n = f - b * HIST
        v = plsc.load_gather(simt_v, [n, b])
        hv = v // HALF
        v = 2 * (v - HALF * hv) + hv
        plsc.store_scatter(sim_v, [f // ROWS, f % ROWS], v)
        return 0

    lax.fori_loop(0, (BW * HIST) // L, build_body, 0)

    def fire(h, rows_v, sem):
        for k in range(DPH):
            pltpu.async_copy(x_h.at[sim_v.at[h * DPH + k]],
                             rows_v.at[pl.ds(k * ROWS, ROWS)], sem)

    fire(0, rows0, sem0)
    fire(1, rows1, sem1)

    one = jnp.ones((L,), jnp.int32)

    def n0_body(lg, _):
        sl = pl.ds(lg * L, L)
        nnz = jnp.zeros((L,), jnp.int32)
        for n in range(HIST):
            nnz = nnz + jnp.minimum(simt_v[n, sl], one)
        n0f_v[sl] = (HIST - nnz).astype(jnp.float32)
        scale_v[sl] = plsc.load_gather(xl_v, [nnz])
        return 0

    lax.fori_loop(0, LG, n0_body, 0)

    cp_ub.wait()
    cp_ib.wait()
    cp_q.wait()
    cp_x0.wait()

    lanes = jnp.arange(L, dtype=jnp.int32)

    def do_half(lg, half, rows_v, sem, n0f16, x0, dotv):
        h = 2 * lg + half
        pltpu.make_async_copy(x_h.at[pl.ds(0, HGE * HIST)], rows_v, sem).wait()

        for p in range(DPH):
            def acc_body(n, a):
                r0 = p * ROWS + 2 * n
                r1 = r0 + HIST
                return tuple(
                    [a[j] + (rows_v[r0, pl.ds(j * L, L)]
                             + rows_v[r0 + 1, pl.ds(j * L, L)])
                     for j in range(NJ)]
                    + [a[NJ + j] + (rows_v[r1, pl.ds(j * L, L)]
                                    + rows_v[r1 + 1, pl.ds(j * L, L)])
                       for j in range(NJ)])

            zero = tuple(jnp.zeros((L,), jnp.float32) for _ in range(2 * NJ))
            a = lax.fori_loop(0, HIST // 2, acc_body, zero)

            for e2 in range(2):
                e = half * HGE + 2 * p + e2
                n0f = n0f16[e]
                s = jnp.zeros((L,), jnp.float32)
                for j in range(NJ):
                    t = a[e2 * NJ + j] - n0f * x0[j]
                    s = s + t * q_v[lg * L + e, pl.ds(j * L, L)]
                oneh = (1 - jnp.minimum(jnp.abs(lanes - e), 1)).astype(
                    jnp.float32)
                dotv = dotv + jnp.sum(s) * oneh

        @pl.when(h + 2 < NH)
        def _():
            fire(h + 2, rows_v, sem)

        return dotv

    def main_body(lg, _):
        sl16 = pl.ds(lg * L, L)
        n0f16 = n0f_v[sl16]
        x0 = [x0r_v[0, pl.ds(j * L, L)] for j in range(NJ)]
        dotv = jnp.zeros((L,), jnp.float32)
        dotv = do_half(lg, 0, rows0, sem0, n0f16, x0, dotv)
        dotv = do_half(lg, 1, rows1, sem1, n0f16, x0, dotv)

        bvec = lg * L + lanes
        ub16 = plsc.load_gather(ubr_v, [bvec, uidx_v[sl16] & 15])
        ib16 = plsc.load_gather(ibr_v, [bvec, iidx_v[sl16] & 15])
        ub_v[sl16] = ub16
        ib_v[sl16] = ib16
        out_v[sl16] = AVG_RATING + ub16 + ib16 + dotv * scale_v[sl16]
        return 0

    lax.fori_loop(0, LG, main_body, 0)

    pltpu.sync_copy(out_v, out_h.at[pl.ds(base, BW)])
    pltpu.sync_copy(ub_v, ubo_h.at[pl.ds(base, BW)])
    pltpu.sync_copy(ib_v, ibo_h.at[pl.ds(base, BW)])


@jax.jit
def kernel(user, item, similar_explicit, user_bias, item_bias, item_q, item_x):
    user = user.astype(jnp.int32)
    item = item.astype(jnp.int32)
    simt = similar_explicit.astype(jnp.int32).T
    lut = 1.0 / (jnp.sqrt(jnp.arange(D, dtype=jnp.float32)) + 1e-13)

    f32, i32 = jnp.float32, jnp.int32
    run = pl.kernel(
        _sc_body,
        out_type=(
            jax.ShapeDtypeStruct((B,), f32),
            jax.ShapeDtypeStruct((B,), f32),
            jax.ShapeDtypeStruct((B,), f32),
        ),
        mesh=plsc.VectorSubcoreMesh(core_axis_name="c", subcore_axis_name="s"),
        compiler_params=pltpu.CompilerParams(use_tc_tiling_on_sc=False,
                                             needs_layout_passes=False),
        scratch_types=(
            pltpu.VMEM((ND, ROWS), i32),
            pltpu.VMEM((HIST, BW), i32),
            pltpu.VMEM((BW,), i32),
            pltpu.VMEM((BW,), i32),
            pltpu.VMEM((BW,), i32),
            pltpu.VMEM((BW,), i32),
            pltpu.VMEM((BW,), i32),
            pltpu.VMEM((BW, L), f32),
            pltpu.VMEM((BW, L), f32),
            pltpu.VMEM((BW,), f32),
            pltpu.VMEM((BW,), f32),
            pltpu.VMEM((BW, D), f32),
            pltpu.VMEM((D,), f32),
            pltpu.VMEM((L,), i32),
            pltpu.VMEM((L, D), f32),
            pltpu.VMEM((BW,), f32),
            pltpu.VMEM((BW,), f32),
            pltpu.VMEM((BW,), f32),
            pltpu.VMEM((HGE * HIST, D), f32),
            pltpu.VMEM((HGE * HIST, D), f32),
            pltpu.SemaphoreType.DMA,
            pltpu.SemaphoreType.DMA,
            pltpu.SemaphoreType.DMA,
        ),
    )
    q_lin = _fold_pairs(item_q).reshape(2 * HALF, D)
    x_lin = _fold_pairs(item_x).reshape(2 * HALF, D)
    out, ub, ib = run(user, item, simt,
                      user_bias.reshape(-1, L), item_bias.reshape(-1, L),
                      q_lin, x_lin, lut)
    return (out, ub, ib)
